# baseline (speedup 1.0000x reference)
import functools

import jax
import jax.numpy as jnp
from jax import lax
from jax.experimental import pallas as pl
from jax.experimental.pallas import tpu as pltpu
from jax.experimental.pallas import tpu_sc as plsc

EMB = 128
ADD = 512
HID = 512
DIN = 2 * EMB + ADD
N_PRED = 26

F32 = jnp.float32
BF16 = jnp.bfloat16


def _init_obj_body(objs_ref, et_ref, bx_ref, emb_ref, w3_ref, b3_ref, out_ref):
    out_ref[:, :ADD] = et_ref[...]
    objs = objs_ref[...]
    ncls = emb_ref.shape[0]
    oh = (objs == lax.broadcasted_iota(jnp.int32, (objs.shape[0], ncls), 1))
    out_ref[:, ADD:ADD + EMB] = jnp.dot(oh.astype(F32), emb_ref[...],
                                        preferred_element_type=F32)
    out_ref[:, ADD + EMB:] = (
        jnp.dot(bx_ref[...], w3_ref[...], preferred_element_type=F32)
        + b3_ref[...])


def _init_obj(objs, enc_text, boxes, emb, w3, b3):
    n = objs.shape[0]
    br = 1000
    ncls = emb.shape[0]
    return pl.pallas_call(
        _init_obj_body,
        grid=(n // br,),
        in_specs=[
            pl.BlockSpec((br, 1), lambda i: (i, 0)),
            pl.BlockSpec((br, ADD), lambda i: (i, 0)),
            pl.BlockSpec((br, 6), lambda i: (i, 0)),
            pl.BlockSpec((ncls, EMB), lambda i: (0, 0)),
            pl.BlockSpec((6, EMB), lambda i: (0, 0)),
            pl.BlockSpec((1, EMB), lambda i: (0, 0)),
        ],
        out_specs=pl.BlockSpec((br, DIN), lambda i: (i, 0)),
        out_shape=jax.ShapeDtypeStruct((n, DIN), F32),
    )(objs.reshape(n, 1).astype(jnp.int32), enc_text, boxes, emb, w3,
      b3.reshape(1, EMB))


def _proj_body(x_ref, w_ref, b_ref, xs_ref, xo_ref):
    y = jnp.dot(x_ref[...], w_ref[...], preferred_element_type=F32)
    xs_ref[...] = (y[:, :HID] + b_ref[...]).astype(BF16)
    xo_ref[...] = y[:, HID:].astype(BF16)


def _proj(x, w_so, b1):
    n = x.shape[0]
    br = 1000
    return pl.pallas_call(
        _proj_body,
        grid=(n // br,),
        in_specs=[
            pl.BlockSpec((br, DIN), lambda i: (i, 0)),
            pl.BlockSpec((DIN, 2 * HID), lambda i: (0, 0)),
            pl.BlockSpec((1, HID), lambda i: (0, 0)),
        ],
        out_specs=[
            pl.BlockSpec((br, HID), lambda i: (i, 0)),
            pl.BlockSpec((br, HID), lambda i: (i, 0)),
        ],
        out_shape=[jax.ShapeDtypeStruct((n, HID), BF16)] * 2,
    )(x, w_so, b1.reshape(1, HID))


def _edge_body(e_ref, gs_ref, go_ref, wp_ref, w2_ref, b2_ref, us_ref,
               up_ref, uo_ref):
    ep = jnp.dot(e_ref[...], wp_ref[...], preferred_element_type=F32)
    t = jnp.maximum(ep + gs_ref[...].astype(F32) + go_ref[...].astype(F32),
                    0.0)
    u = jnp.dot(t, w2_ref[...], preferred_element_type=F32) + b2_ref[...]
    u = jnp.maximum(u, 0.0)
    us_ref[...] = u[:, :HID]
    up_ref[...] = u[:, HID:HID + DIN]
    uo_ref[...] = u[:, HID + DIN:]


def _edge(e, gs, go, wp, w2, b2):
    n = e.shape[0]
    br = 640
    dout = 2 * HID + DIN
    return pl.pallas_call(
        _edge_body,
        grid=(n // br,),
        in_specs=[
            pl.BlockSpec((br, DIN), lambda i: (i, 0)),
            pl.BlockSpec((br, HID), lambda i: (i, 0)),
            pl.BlockSpec((br, HID), lambda i: (i, 0)),
            pl.BlockSpec((DIN, HID), lambda i: (0, 0)),
            pl.BlockSpec((HID, dout), lambda i: (0, 0)),
            pl.BlockSpec((1, dout), lambda i: (0, 0)),
        ],
        out_specs=[
            pl.BlockSpec((br, HID), lambda i: (i, 0)),
            pl.BlockSpec((br, DIN), lambda i: (i, 0)),
            pl.BlockSpec((br, HID), lambda i: (i, 0)),
        ],
        out_shape=[
            jax.ShapeDtypeStruct((n, HID), F32),
            jax.ShapeDtypeStruct((n, DIN), F32),
            jax.ShapeDtypeStruct((n, HID), F32),
        ],
    )(e, gs, go, wp, w2, b2.reshape(1, dout))


def _edge1_body(er_ref, p_ref, gs_ref, go_ref, wpa_ref, pemb_ref, wpb_ref,
                w2_ref, b2_ref, us_ref, up_ref, uo_ref):
    ep = jnp.dot(er_ref[...], wpa_ref[...], preferred_element_type=F32)
    tab = jnp.dot(pemb_ref[...], wpb_ref[...], preferred_element_type=F32)
    p = p_ref[...]
    oh = (p == lax.broadcasted_iota(jnp.int32, (p.shape[0], N_PRED), 1))
    ep = ep + jnp.dot(oh.astype(F32), tab, preferred_element_type=F32)
    t = jnp.maximum(ep + gs_ref[...].astype(F32) + go_ref[...].astype(F32),
                    0.0)
    u = jnp.dot(t, w2_ref[...], preferred_element_type=F32) + b2_ref[...]
    u = jnp.maximum(u, 0.0)
    us_ref[...] = u[:, :HID]
    up_ref[...] = u[:, HID:HID + DIN]
    uo_ref[...] = u[:, HID + DIN:]


def _edge1(enc_rel, p, gs, go, wpa, pemb, wpb, w2, b2):
    n = enc_rel.shape[0]
    br = 640
    dout = 2 * HID + DIN
    return pl.pallas_call(
        _edge1_body,
        grid=(n // br,),
        in_specs=[
            pl.BlockSpec((br, ADD), lambda i: (i, 0)),
            pl.BlockSpec((br, 1), lambda i: (i, 0)),
            pl.BlockSpec((br, HID), lambda i: (i, 0)),
            pl.BlockSpec((br, HID), lambda i: (i, 0)),
            pl.BlockSpec((ADD, HID), lambda i: (0, 0)),
            pl.BlockSpec((N_PRED, 2 * EMB), lambda i: (0, 0)),
            pl.BlockSpec((2 * EMB, HID), lambda i: (0, 0)),
            pl.BlockSpec((HID, dout), lambda i: (0, 0)),
            pl.BlockSpec((1, dout), lambda i: (0, 0)),
        ],
        out_specs=[
            pl.BlockSpec((br, HID), lambda i: (i, 0)),
            pl.BlockSpec((br, DIN), lambda i: (i, 0)),
            pl.BlockSpec((br, HID), lambda i: (i, 0)),
        ],
        out_shape=[
            jax.ShapeDtypeStruct((n, HID), F32),
            jax.ShapeDtypeStruct((n, DIN), F32),
            jax.ShapeDtypeStruct((n, HID), F32),
        ],
    )(enc_rel, p.reshape(n, 1).astype(jnp.int32), gs, go, wpa, pemb, wpb,
      w2, b2.reshape(1, dout))


def _edge_last_body(e_ref, gs_ref, go_ref, wp_ref, w2_ref, b2_ref, us_ref,
                    uo_ref):
    ep = jnp.dot(e_ref[...], wp_ref[...], preferred_element_type=F32)
    t = jnp.maximum(ep + gs_ref[...].astype(F32) + go_ref[...].astype(F32),
                    0.0)
    u = jnp.dot(t, w2_ref[...], preferred_element_type=F32) + b2_ref[...]
    u = jnp.maximum(u, 0.0)
    us_ref[...] = u[:, :HID]
    uo_ref[...] = u[:, HID:]


def _edge_last(e, gs, go, wp, w2so, b2so):
    n = e.shape[0]
    br = 640
    return pl.pallas_call(
        _edge_last_body,
        grid=(n // br,),
        in_specs=[
            pl.BlockSpec((br, DIN), lambda i: (i, 0)),
            pl.BlockSpec((br, HID), lambda i: (i, 0)),
            pl.BlockSpec((br, HID), lambda i: (i, 0)),
            pl.BlockSpec((DIN, HID), lambda i: (0, 0)),
            pl.BlockSpec((HID, 2 * HID), lambda i: (0, 0)),
            pl.BlockSpec((1, 2 * HID), lambda i: (0, 0)),
        ],
        out_specs=[
            pl.BlockSpec((br, HID), lambda i: (i, 0)),
            pl.BlockSpec((br, HID), lambda i: (i, 0)),
        ],
        out_shape=[jax.ShapeDtypeStruct((n, HID), F32)] * 2,
    )(e, gs, go, wp, w2so, b2so.reshape(1, 2 * HID))


def _node_body(pool_ref, cnt_ref, v1_ref, c1_ref, v2_ref, c2_ref, out_ref):
    cnt = cnt_ref[0, :, :1] + cnt_ref[1, :, :1]
    inv = 1.0 / jnp.maximum(cnt, 1.0)
    x = pool_ref[...] * inv
    h = jnp.maximum(
        jnp.dot(x, v1_ref[...], preferred_element_type=F32) + c1_ref[...], 0.0)
    out_ref[...] = jnp.maximum(
        jnp.dot(h, v2_ref[...], preferred_element_type=F32) + c2_ref[...], 0.0)


def _node(pooled, counts2, v1, c1, v2, c2, n):
    br = 1000
    return pl.pallas_call(
        _node_body,
        grid=(n // br,),
        in_specs=[
            pl.BlockSpec((br, HID), lambda i: (i, 0)),
            pl.BlockSpec((2, br, 128), lambda i: (0, i, 0)),
            pl.BlockSpec((HID, HID), lambda i: (0, 0)),
            pl.BlockSpec((1, HID), lambda i: (0, 0)),
            pl.BlockSpec((HID, DIN), lambda i: (0, 0)),
            pl.BlockSpec((1, DIN), lambda i: (0, 0)),
        ],
        out_specs=pl.BlockSpec((br, DIN), lambda i: (i, 0)),
        out_shape=jax.ShapeDtypeStruct((n, DIN), F32),
    )(pooled, counts2, v1, c1.reshape(1, HID), v2, c2.reshape(1, DIN))


def _heads_body(x_ref, m1_ref, d1_ref, m2_ref, d2_ref, wm_ref, bm_ref,
                wv_ref, bv_ref, mu_ref, lv_ref):
    h = jnp.maximum(
        jnp.dot(x_ref[...], m1_ref[...], preferred_element_type=F32)
        + d1_ref[...], 0.0)
    ov3 = jnp.maximum(
        jnp.dot(h, m2_ref[...], preferred_element_type=F32) + d2_ref[...], 0.0)
    mu_ref[...] = (jnp.dot(ov3, wm_ref[...], preferred_element_type=F32)
                   + bm_ref[...])
    lv_ref[...] = (jnp.dot(ov3, wv_ref[...], preferred_element_type=F32)
                   + bv_ref[...])


def _heads(x, mean_var, mean, var):
    n = x.shape[0]
    br = 1000
    (m1, d1), (m2, d2) = mean_var
    (wm, bm), = mean
    (wv, bv), = var
    return pl.pallas_call(
        _heads_body,
        grid=(n // br,),
        in_specs=[
            pl.BlockSpec((br, DIN), lambda i: (i, 0)),
            pl.BlockSpec((DIN, 4 * EMB), lambda i: (0, 0)),
            pl.BlockSpec((1, 4 * EMB), lambda i: (0, 0)),
            pl.BlockSpec((4 * EMB, 2 * EMB), lambda i: (0, 0)),
            pl.BlockSpec((1, 2 * EMB), lambda i: (0, 0)),
            pl.BlockSpec((2 * EMB, EMB), lambda i: (0, 0)),
            pl.BlockSpec((1, EMB), lambda i: (0, 0)),
            pl.BlockSpec((2 * EMB, EMB), lambda i: (0, 0)),
            pl.BlockSpec((1, EMB), lambda i: (0, 0)),
        ],
        out_specs=[
            pl.BlockSpec((br, EMB), lambda i: (i, 0)),
            pl.BlockSpec((br, EMB), lambda i: (i, 0)),
        ],
        out_shape=[jax.ShapeDtypeStruct((n, EMB), F32)] * 2,
    )(x, m1, d1.reshape(1, 4 * EMB), m2, d2.reshape(1, 2 * EMB),
      wm, bm.reshape(1, EMB), wv, bv.reshape(1, EMB))


_GCH = 40
_SCH = 80
_ZR = 128


def _pad_nodes(n_obj, ns):
    q = ns * _ZR
    return ((n_obj + q - 1) // q) * q


_GW = HID // 2


def _sc_gather(xs, xo, s_idx, o_idx):
    n_edge = s_idx.shape[0]
    info = plsc.get_sparse_core_info()
    nc, ns = info.num_cores, info.num_subcores
    nw = nc * ns
    per_w = n_edge // nw
    n_ch = per_w // _GCH
    assert per_w % _GCH == 0 and per_w % 8 == 0 and n_ch >= 4
    npairs = (n_ch - 3) // 2
    mesh = plsc.VectorSubcoreMesh(core_axis_name="c", subcore_axis_name="s")

    @functools.partial(
        pl.kernel,
        out_type=[jax.ShapeDtypeStruct((n_edge, _GW), jnp.int32)] * 2,
        mesh=mesh,
        scratch_types=[
            pltpu.VMEM((per_w,), jnp.int32),
            pltpu.VMEM((per_w,), jnp.int32),
            pltpu.VMEM((_GCH, _GW), jnp.int32),
            pltpu.VMEM((_GCH, _GW), jnp.int32),
            pltpu.VMEM((_GCH, _GW), jnp.int32),
            pltpu.VMEM((_GCH, _GW), jnp.int32),
            pltpu.SemaphoreType.DMA,
            pltpu.SemaphoreType.DMA,
            pltpu.SemaphoreType.DMA,
            pltpu.SemaphoreType.DMA,
            pltpu.SemaphoreType.DMA,
            pltpu.SemaphoreType.DMA,
            pltpu.SemaphoreType.DMA,
            pltpu.SemaphoreType.DMA,
        ],
    )
    def k(xs_hbm, xo_hbm, s_hbm, o_hbm, outs_hbm, outo_hbm, idx_s, idx_o,
          bs0, bo0, bs1, bo1, sgs0, sgo0, sgs1, sgo1, sws0, swo0, sws1, swo1):
        wid = lax.axis_index("s") * nc + lax.axis_index("c")
        base = wid * per_w
        pltpu.sync_copy(s_hbm.at[pl.ds(base, per_w)], idx_s)
        pltpu.sync_copy(o_hbm.at[pl.ds(base, per_w)], idx_o)

        bufs = ((bs0, bo0, sgs0, sgo0, sws0, swo0),
                (bs1, bo1, sgs1, sgo1, sws1, swo1))

        def start(c, b):
            bs, bo, sgs, sgo, _, _ = bufs[b]
            off = c * _GCH
            pltpu.async_copy(xs_hbm.at[idx_s.at[pl.ds(off, _GCH)]], bs, sgs)
            pltpu.async_copy(xo_hbm.at[idx_o.at[pl.ds(off, _GCH)]], bo, sgo)

        def compute(c, b):
            bs, bo, sgs, sgo, sws, swo = bufs[b]
            off = c * _GCH
            pltpu.make_async_copy(
                xs_hbm.at[idx_s.at[pl.ds(off, _GCH)]], bs, sgs).wait()
            pltpu.make_async_copy(
                xo_hbm.at[idx_o.at[pl.ds(off, _GCH)]], bo, sgo).wait()
            pltpu.async_copy(bs, outs_hbm.at[pl.ds(base + off, _GCH)], sws)
            pltpu.async_copy(bo, outo_hbm.at[pl.ds(base + off, _GCH)], swo)

        def drain_w(c, b):
            bs, bo, _, _, sws, swo = bufs[b]
            off = base + c * _GCH
            pltpu.make_async_copy(
                bs, outs_hbm.at[pl.ds(off, _GCH)], sws).wait()
            pltpu.make_async_copy(
                bo, outo_hbm.at[pl.ds(off, _GCH)], swo).wait()

        start(0, 0)
        start(1, 1)

        def pair(j, carry):
            c0 = 2 * j
            compute(c0, 0)
            compute(c0 + 1, 1)
            drain_w(c0, 0)
            start(c0 + 2, 0)
            drain_w(c0 + 1, 1)
            start(c0 + 3, 1)
            return carry

        lax.fori_loop(0, npairs, pair, 0)
        for c in range(2 * npairs, n_ch):
            b = c % 2
            compute(c, b)
            drain_w(c, b)
            if c + 2 < n_ch:
                start(c + 2, b)

    return k(xs, xo, s_idx, o_idx)


def _sc_scatter(us, uo, s_idx, o_idx, n_obj):
    n_edge = s_idx.shape[0]
    info = plsc.get_sparse_core_info()
    nc, ns = info.num_cores, info.num_subcores
    per_t = n_edge // ns
    n_ch = per_t // _SCH
    n_pad = _pad_nodes(n_obj, ns)
    stripe = n_pad // ns
    nz = stripe // _ZR
    n_sl = HID // 128
    sl_per_core = n_sl // nc
    assert per_t % _SCH == 0 and stripe % _ZR == 0
    mesh = plsc.VectorSubcoreMesh(core_axis_name="c", subcore_axis_name="s")

    assert n_ch >= 4
    npairs = (n_ch - 3) // 2

    @functools.partial(
        pl.kernel,
        out_type=jax.ShapeDtypeStruct((n_pad, HID), F32),
        mesh=mesh,
        scratch_types=[
            pltpu.VMEM((_SCH,), jnp.int32),
            pltpu.VMEM((_SCH,), jnp.int32),
            pltpu.VMEM((_SCH, 128), F32),
            pltpu.VMEM((_SCH, 128), F32),
            pltpu.VMEM((_ZR, 128), F32),
            pltpu.VMEM_SHARED((n_pad, 128), F32),
            pltpu.SemaphoreType.DMA,
            pltpu.SemaphoreType.DMA,
            pltpu.SemaphoreType.DMA,
            pltpu.SemaphoreType.DMA,
            pltpu.SemaphoreType.DMA,
            pltpu.SemaphoreType.DMA,
        ],
    )
    def k(us_hbm, uo_hbm, s_hbm, o_hbm, out_hbm, ib0, ib1, r0, r1,
          zero_b, acc, si0, si1, sr0, sr1, sa0, sa1):
        cid = lax.axis_index("c")
        sid = lax.axis_index("s")

        def zrow(r, c2):
            for cc in range(128 // 16):
                zero_b[r, pl.ds(cc * 16, 16)] = jnp.zeros((16,), F32)
            return c2

        lax.fori_loop(0, _ZR, zrow, 0)

        for f_local in range(sl_per_core):
            f = cid * sl_per_core + f_local
            for z in range(nz):
                pltpu.sync_copy(zero_b,
                                acc.at[pl.ds(sid * stripe + z * _ZR, _ZR)])
            plsc.subcore_barrier()

            for src_hbm, idx_hbm in ((us_hbm, s_hbm), (uo_hbm, o_hbm)):
                bufs = ((ib0, r0, si0, sr0, sa0), (ib1, r1, si1, sr1, sa1))

                def start(c, b):
                    ib, rb, si, sr, _ = bufs[b]
                    off = sid * per_t + c * _SCH
                    pltpu.async_copy(idx_hbm.at[pl.ds(off, _SCH)], ib, si)
                    pltpu.async_copy(
                        src_hbm.at[pl.ds(off, _SCH), pl.ds(f * 128, 128)],
                        rb, sr)

                def scat(c, b):
                    ib, rb, si, sr, sa = bufs[b]
                    off = sid * per_t + c * _SCH
                    pltpu.make_async_copy(
                        idx_hbm.at[pl.ds(off, _SCH)], ib, si).wait()
                    pltpu.make_async_copy(
                        src_hbm.at[pl.ds(off, _SCH), pl.ds(f * 128, 128)],
                        rb, sr).wait()
                    pltpu.async_copy(rb, acc.at[ib], sa, add=True)

                def drain_a(c, b):
                    ib, rb, _, _, sa = bufs[b]
                    pltpu.make_async_copy(rb, acc.at[ib], sa).wait()

                start(0, 0)
                start(1, 1)

                def pair(j, carry):
                    c0 = 2 * j
                    scat(c0, 0)
                    scat(c0 + 1, 1)
                    drain_a(c0, 0)
                    start(c0 + 2, 0)
                    drain_a(c0 + 1, 1)
                    start(c0 + 3, 1)
                    return carry

                lax.fori_loop(0, npairs, pair, 0)
                for c in range(2 * npairs, n_ch):
                    b = c % 2
                    scat(c, b)
                    drain_a(c, b)
                    if c + 2 < n_ch:
                        start(c + 2, b)

            plsc.subcore_barrier()
            pltpu.sync_copy(
                acc.at[pl.ds(sid * stripe, stripe)],
                out_hbm.at[pl.ds(sid * stripe, stripe), pl.ds(f * 128, 128)])
            plsc.subcore_barrier()

    return k(us, uo, s_idx, o_idx)


def _sc_counts(s_idx, o_idx, n_obj):
    n_edge = s_idx.shape[0]
    info = plsc.get_sparse_core_info()
    nc, ns = info.num_cores, info.num_subcores
    per_t = n_edge // ns
    n_ch = per_t // _SCH
    n_pad = _pad_nodes(n_obj, ns)
    stripe = n_pad // ns
    nz = stripe // _ZR
    assert per_t % _SCH == 0 and stripe % _ZR == 0 and nc == 2
    mesh = plsc.VectorSubcoreMesh(core_axis_name="c", subcore_axis_name="s")

    @functools.partial(
        pl.kernel,
        out_type=jax.ShapeDtypeStruct((nc * n_pad, 128), F32),
        mesh=mesh,
        scratch_types=[
            pltpu.VMEM((_SCH,), jnp.int32),
            pltpu.VMEM((_SCH, 128), F32),
            pltpu.VMEM((_ZR, 128), F32),
            pltpu.VMEM_SHARED((n_pad, 128), F32),
        ],
    )
    def k(s_hbm, o_hbm, out_hbm, idx_b, ones_b, zero_b, acc):
        cid = lax.axis_index("c")
        sid = lax.axis_index("s")

        def onerow(r, c2):
            for cc in range(128 // 16):
                ones_b[r, pl.ds(cc * 16, 16)] = jnp.ones((16,), F32)
            return c2

        lax.fori_loop(0, _SCH, onerow, 0)

        def zrow(r, c2):
            for cc in range(128 // 16):
                zero_b[r, pl.ds(cc * 16, 16)] = jnp.zeros((16,), F32)
            return c2

        lax.fori_loop(0, _ZR, zrow, 0)
        for z in range(nz):
            pltpu.sync_copy(zero_b, acc.at[pl.ds(sid * stripe + z * _ZR, _ZR)])
        plsc.subcore_barrier()

        def chunk(i, carry, idx_hbm):
            off = sid * per_t + i * _SCH
            pltpu.sync_copy(idx_hbm.at[pl.ds(off, _SCH)], idx_b)
            pltpu.sync_copy(ones_b, acc.at[idx_b], add=True)
            return carry

        @pl.when(cid == 0)
        def _():
            lax.fori_loop(0, n_ch, functools.partial(chunk, idx_hbm=s_hbm), 0)

        @pl.when(cid == 1)
        def _():
            lax.fori_loop(0, n_ch, functools.partial(chunk, idx_hbm=o_hbm), 0)

        plsc.subcore_barrier()
        pltpu.sync_copy(
            acc.at[pl.ds(sid * stripe, stripe)],
            out_hbm.at[pl.ds(cid * n_pad + sid * stripe, stripe)])

    return k(s_idx, o_idx)


def kernel(objs, triples, boxes_gt, attributes, enc_text_feat, enc_rel_feat,
           params):
    del attributes
    n_obj = objs.shape[0]
    s = triples[:, 0].astype(jnp.int32)
    p = triples[:, 1].astype(jnp.int32)
    o = triples[:, 2].astype(jnp.int32)

    w3, b3 = params["d3"]
    obj = _init_obj(objs, enc_text_feat, boxes_gt, params["obj_emb"], w3, b3)

    counts_flat = _sc_counts(s, o, n_obj)
    counts2 = counts_flat.reshape(2, -1, 128)

    e_feat = None
    for li, layer in enumerate(params["gconv"]):
        (w1, b1), (w2, b2) = layer["net1"]
        (v1, c1), (v2, c2) = layer["net2"]
        ws = w1[:DIN]
        wp = w1[DIN:2 * DIN]
        wo = w1[2 * DIN:]
        w_so = jnp.concatenate([ws, wo], axis=1)

        xsb, xo = _proj(obj, w_so, b1)
        gs_i, go_i = _sc_gather(
            lax.bitcast_convert_type(xsb.reshape(n_obj, _GW, 2), jnp.int32),
            lax.bitcast_convert_type(xo.reshape(n_obj, _GW, 2), jnp.int32),
            s, o)
        gs = lax.bitcast_convert_type(gs_i, BF16).reshape(-1, HID)
        go = lax.bitcast_convert_type(go_i, BF16).reshape(-1, HID)

        if li == 0:
            us, up, uo = _edge1(enc_rel_feat, p, gs, go, wp[:ADD],
                                params["pred_emb"], wp[ADD:], w2, b2)
        elif li == len(params["gconv"]) - 1:
            w2so = jnp.concatenate([w2[:, :HID], w2[:, HID + DIN:]], axis=1)
            b2so = jnp.concatenate([b2[:HID], b2[HID + DIN:]])
            us, uo = _edge_last(e_feat, gs, go, wp, w2so, b2so)
            up = None
        else:
            us, up, uo = _edge(e_feat, gs, go, wp, w2, b2)

        pooled = _sc_scatter(us, uo, s, o, n_obj)
        obj = _node(pooled, counts2, v1, c1, v2, c2, n_obj)
        e_feat = up

    mu, logvar = _heads(obj, params["mean_var"], params["mean"], params["var"])
    return mu, logvar


# speedup vs baseline: 2.9022x; 2.9022x over previous
import functools

import jax
import jax.numpy as jnp
from jax import lax
from jax.experimental import pallas as pl
from jax.experimental.pallas import tpu as pltpu
from jax.experimental.pallas import tpu_sc as plsc

EMB = 128
ADD = 512
HID = 512
DIN = 2 * EMB + ADD
N_PRED = 26

F32 = jnp.float32
BF16 = jnp.bfloat16
U32 = jnp.uint32
_GW = HID // 2


def _pack_bf16(v):
    lo = lax.bitcast_convert_type(v[:, :_GW], U32)
    hi = lax.bitcast_convert_type(v[:, _GW:], U32)

    def rne(u):
        return u + jnp.asarray(0x7FFF, U32) + ((u >> 16) & jnp.asarray(1, U32))

    packed = (rne(hi) & jnp.asarray(0xFFFF0000, U32)) | (rne(lo) >> 16)
    return lax.bitcast_convert_type(packed, jnp.int32)


def _unpack_bf16(gi):
    u = lax.bitcast_convert_type(gi, U32)
    lo = lax.bitcast_convert_type(u << 16, F32)
    hi = lax.bitcast_convert_type(u & jnp.asarray(0xFFFF0000, U32), F32)
    return jnp.concatenate([lo, hi], axis=1)


def _init_obj_body(objs_ref, et_ref, bx_ref, emb_ref, w3_ref, b3_ref, out_ref):
    out_ref[:, :ADD] = et_ref[...]
    objs = objs_ref[...]
    ncls = emb_ref.shape[0]
    oh = (objs == lax.broadcasted_iota(jnp.int32, (objs.shape[0], ncls), 1))
    out_ref[:, ADD:ADD + EMB] = jnp.dot(oh.astype(F32), emb_ref[...],
                                        preferred_element_type=F32)
    out_ref[:, ADD + EMB:] = (
        jnp.dot(bx_ref[...], w3_ref[...], preferred_element_type=F32)
        + b3_ref[...])


def _init_obj(objs, enc_text, boxes, emb, w3, b3):
    n = objs.shape[0]
    br = 1000
    ncls = emb.shape[0]
    return pl.pallas_call(
        _init_obj_body,
        grid=(n // br,),
        in_specs=[
            pl.BlockSpec((br, 1), lambda i: (i, 0)),
            pl.BlockSpec((br, ADD), lambda i: (i, 0)),
            pl.BlockSpec((br, 6), lambda i: (i, 0)),
            pl.BlockSpec((ncls, EMB), lambda i: (0, 0)),
            pl.BlockSpec((6, EMB), lambda i: (0, 0)),
            pl.BlockSpec((1, EMB), lambda i: (0, 0)),
        ],
        out_specs=pl.BlockSpec((br, DIN), lambda i: (i, 0)),
        out_shape=jax.ShapeDtypeStruct((n, DIN), F32),
    )(objs.reshape(n, 1).astype(jnp.int32), enc_text, boxes, emb, w3,
      b3.reshape(1, EMB))


def _proj_body(x_ref, w_ref, b_ref, xs_ref, xo_ref):
    y = jnp.dot(x_ref[...], w_ref[...], preferred_element_type=F32)
    xs_ref[...] = _pack_bf16(y[:, :HID] + b_ref[...])
    xo_ref[...] = _pack_bf16(y[:, HID:])


def _proj(x, w_so, b1):
    n = x.shape[0]
    br = 1000
    return pl.pallas_call(
        _proj_body,
        grid=(n // br,),
        in_specs=[
            pl.BlockSpec((br, DIN), lambda i: (i, 0)),
            pl.BlockSpec((DIN, 2 * HID), lambda i: (0, 0)),
            pl.BlockSpec((1, HID), lambda i: (0, 0)),
        ],
        out_specs=[
            pl.BlockSpec((br, _GW), lambda i: (i, 0)),
            pl.BlockSpec((br, _GW), lambda i: (i, 0)),
        ],
        out_shape=[jax.ShapeDtypeStruct((n, _GW), jnp.int32)] * 2,
    )(x, w_so, b1.reshape(1, HID))


def _edge_body(e_ref, gs_ref, go_ref, wp_ref, w2_ref, b2_ref, us_ref,
               up_ref, uo_ref):
    ep = jnp.dot(e_ref[...], wp_ref[...], preferred_element_type=F32)
    t = jnp.maximum(
        ep + _unpack_bf16(gs_ref[...]) + _unpack_bf16(go_ref[...]), 0.0)
    u = jnp.dot(t, w2_ref[...], preferred_element_type=F32) + b2_ref[...]
    u = jnp.maximum(u, 0.0)
    us_ref[...] = u[:, :HID]
    up_ref[...] = u[:, HID:HID + DIN]
    uo_ref[...] = u[:, HID + DIN:]


def _edge(e, gs, go, wp, w2, b2):
    n = e.shape[0]
    br = 640
    dout = 2 * HID + DIN
    return pl.pallas_call(
        _edge_body,
        grid=(n // br,),
        in_specs=[
            pl.BlockSpec((br, DIN), lambda i: (i, 0)),
            pl.BlockSpec((br, _GW), lambda i: (i, 0)),
            pl.BlockSpec((br, _GW), lambda i: (i, 0)),
            pl.BlockSpec((DIN, HID), lambda i: (0, 0)),
            pl.BlockSpec((HID, dout), lambda i: (0, 0)),
            pl.BlockSpec((1, dout), lambda i: (0, 0)),
        ],
        out_specs=[
            pl.BlockSpec((br, HID), lambda i: (i, 0)),
            pl.BlockSpec((br, DIN), lambda i: (i, 0)),
            pl.BlockSpec((br, HID), lambda i: (i, 0)),
        ],
        out_shape=[
            jax.ShapeDtypeStruct((n, HID), F32),
            jax.ShapeDtypeStruct((n, DIN), F32),
            jax.ShapeDtypeStruct((n, HID), F32),
        ],
    )(e, gs, go, wp, w2, b2.reshape(1, dout))


def _edge1_body(er_ref, p_ref, gs_ref, go_ref, wpa_ref, pemb_ref, wpb_ref,
                w2_ref, b2_ref, us_ref, up_ref, uo_ref):
    ep = jnp.dot(er_ref[...], wpa_ref[...], preferred_element_type=F32)
    tab = jnp.dot(pemb_ref[...], wpb_ref[...], preferred_element_type=F32)
    p = p_ref[...]
    oh = (p == lax.broadcasted_iota(jnp.int32, (p.shape[0], N_PRED), 1))
    ep = ep + jnp.dot(oh.astype(F32), tab, preferred_element_type=F32)
    t = jnp.maximum(
        ep + _unpack_bf16(gs_ref[...]) + _unpack_bf16(go_ref[...]), 0.0)
    u = jnp.dot(t, w2_ref[...], preferred_element_type=F32) + b2_ref[...]
    u = jnp.maximum(u, 0.0)
    us_ref[...] = u[:, :HID]
    up_ref[...] = u[:, HID:HID + DIN]
    uo_ref[...] = u[:, HID + DIN:]


def _edge1(enc_rel, p, gs, go, wpa, pemb, wpb, w2, b2):
    n = enc_rel.shape[0]
    br = 640
    dout = 2 * HID + DIN
    return pl.pallas_call(
        _edge1_body,
        grid=(n // br,),
        in_specs=[
            pl.BlockSpec((br, ADD), lambda i: (i, 0)),
            pl.BlockSpec((br, 1), lambda i: (i, 0)),
            pl.BlockSpec((br, _GW), lambda i: (i, 0)),
            pl.BlockSpec((br, _GW), lambda i: (i, 0)),
            pl.BlockSpec((ADD, HID), lambda i: (0, 0)),
            pl.BlockSpec((N_PRED, 2 * EMB), lambda i: (0, 0)),
            pl.BlockSpec((2 * EMB, HID), lambda i: (0, 0)),
            pl.BlockSpec((HID, dout), lambda i: (0, 0)),
            pl.BlockSpec((1, dout), lambda i: (0, 0)),
        ],
        out_specs=[
            pl.BlockSpec((br, HID), lambda i: (i, 0)),
            pl.BlockSpec((br, DIN), lambda i: (i, 0)),
            pl.BlockSpec((br, HID), lambda i: (i, 0)),
        ],
        out_shape=[
            jax.ShapeDtypeStruct((n, HID), F32),
            jax.ShapeDtypeStruct((n, DIN), F32),
            jax.ShapeDtypeStruct((n, HID), F32),
        ],
    )(enc_rel, p.reshape(n, 1).astype(jnp.int32), gs, go, wpa, pemb, wpb,
      w2, b2.reshape(1, dout))


def _edge_last_body(e_ref, gs_ref, go_ref, wp_ref, w2_ref, b2_ref, us_ref,
                    uo_ref):
    ep = jnp.dot(e_ref[...], wp_ref[...], preferred_element_type=F32)
    t = jnp.maximum(
        ep + _unpack_bf16(gs_ref[...]) + _unpack_bf16(go_ref[...]), 0.0)
    u = jnp.dot(t, w2_ref[...], preferred_element_type=F32) + b2_ref[...]
    u = jnp.maximum(u, 0.0)
    us_ref[...] = u[:, :HID]
    uo_ref[...] = u[:, HID:]


def _edge_last(e, gs, go, wp, w2so, b2so):
    n = e.shape[0]
    br = 640
    return pl.pallas_call(
        _edge_last_body,
        grid=(n // br,),
        in_specs=[
            pl.BlockSpec((br, DIN), lambda i: (i, 0)),
            pl.BlockSpec((br, _GW), lambda i: (i, 0)),
            pl.BlockSpec((br, _GW), lambda i: (i, 0)),
            pl.BlockSpec((DIN, HID), lambda i: (0, 0)),
            pl.BlockSpec((HID, 2 * HID), lambda i: (0, 0)),
            pl.BlockSpec((1, 2 * HID), lambda i: (0, 0)),
        ],
        out_specs=[
            pl.BlockSpec((br, HID), lambda i: (i, 0)),
            pl.BlockSpec((br, HID), lambda i: (i, 0)),
        ],
        out_shape=[jax.ShapeDtypeStruct((n, HID), F32)] * 2,
    )(e, gs, go, wp, w2so, b2so.reshape(1, 2 * HID))


def _node_body(pool_ref, cnt_ref, v1_ref, c1_ref, v2_ref, c2_ref, out_ref):
    cnt = cnt_ref[0, :, :1] + cnt_ref[1, :, :1]
    inv = 1.0 / jnp.maximum(cnt, 1.0)
    x = pool_ref[...] * inv
    h = jnp.maximum(
        jnp.dot(x, v1_ref[...], preferred_element_type=F32) + c1_ref[...], 0.0)
    out_ref[...] = jnp.maximum(
        jnp.dot(h, v2_ref[...], preferred_element_type=F32) + c2_ref[...], 0.0)


def _node(pooled, counts2, v1, c1, v2, c2, n):
    br = 1000
    return pl.pallas_call(
        _node_body,
        grid=(n // br,),
        in_specs=[
            pl.BlockSpec((br, HID), lambda i: (i, 0)),
            pl.BlockSpec((2, br, 128), lambda i: (0, i, 0)),
            pl.BlockSpec((HID, HID), lambda i: (0, 0)),
            pl.BlockSpec((1, HID), lambda i: (0, 0)),
            pl.BlockSpec((HID, DIN), lambda i: (0, 0)),
            pl.BlockSpec((1, DIN), lambda i: (0, 0)),
        ],
        out_specs=pl.BlockSpec((br, DIN), lambda i: (i, 0)),
        out_shape=jax.ShapeDtypeStruct((n, DIN), F32),
    )(pooled, counts2, v1, c1.reshape(1, HID), v2, c2.reshape(1, DIN))


def _heads_body(x_ref, m1_ref, d1_ref, m2_ref, d2_ref, wm_ref, bm_ref,
                wv_ref, bv_ref, mu_ref, lv_ref):
    h = jnp.maximum(
        jnp.dot(x_ref[...], m1_ref[...], preferred_element_type=F32)
        + d1_ref[...], 0.0)
    ov3 = jnp.maximum(
        jnp.dot(h, m2_ref[...], preferred_element_type=F32) + d2_ref[...], 0.0)
    mu_ref[...] = (jnp.dot(ov3, wm_ref[...], preferred_element_type=F32)
                   + bm_ref[...])
    lv_ref[...] = (jnp.dot(ov3, wv_ref[...], preferred_element_type=F32)
                   + bv_ref[...])


def _heads(x, mean_var, mean, var):
    n = x.shape[0]
    br = 1000
    (m1, d1), (m2, d2) = mean_var
    (wm, bm), = mean
    (wv, bv), = var
    return pl.pallas_call(
        _heads_body,
        grid=(n // br,),
        in_specs=[
            pl.BlockSpec((br, DIN), lambda i: (i, 0)),
            pl.BlockSpec((DIN, 4 * EMB), lambda i: (0, 0)),
            pl.BlockSpec((1, 4 * EMB), lambda i: (0, 0)),
            pl.BlockSpec((4 * EMB, 2 * EMB), lambda i: (0, 0)),
            pl.BlockSpec((1, 2 * EMB), lambda i: (0, 0)),
            pl.BlockSpec((2 * EMB, EMB), lambda i: (0, 0)),
            pl.BlockSpec((1, EMB), lambda i: (0, 0)),
            pl.BlockSpec((2 * EMB, EMB), lambda i: (0, 0)),
            pl.BlockSpec((1, EMB), lambda i: (0, 0)),
        ],
        out_specs=[
            pl.BlockSpec((br, EMB), lambda i: (i, 0)),
            pl.BlockSpec((br, EMB), lambda i: (i, 0)),
        ],
        out_shape=[jax.ShapeDtypeStruct((n, EMB), F32)] * 2,
    )(x, m1, d1.reshape(1, 4 * EMB), m2, d2.reshape(1, 2 * EMB),
      wm, bm.reshape(1, EMB), wv, bv.reshape(1, EMB))


_GCH = 40
_SCH = 80
_ZR = 128


def _pad_nodes(n_obj, ns):
    q = ns * _ZR
    return ((n_obj + q - 1) // q) * q


def _sc_gather(xs, xo, s_idx, o_idx):
    n_edge = s_idx.shape[0]
    info = plsc.get_sparse_core_info()
    nc, ns = info.num_cores, info.num_subcores
    nw = nc * ns
    per_w = n_edge // nw
    n_ch = per_w // _GCH
    assert per_w % _GCH == 0 and per_w % 8 == 0 and n_ch >= 4
    npairs = (n_ch - 3) // 2
    mesh = plsc.VectorSubcoreMesh(core_axis_name="c", subcore_axis_name="s")

    @functools.partial(
        pl.kernel,
        out_type=[jax.ShapeDtypeStruct((n_edge, _GW), jnp.int32)] * 2,
        mesh=mesh,
        scratch_types=[
            pltpu.VMEM((per_w,), jnp.int32),
            pltpu.VMEM((per_w,), jnp.int32),
            pltpu.VMEM((_GCH, _GW), jnp.int32),
            pltpu.VMEM((_GCH, _GW), jnp.int32),
            pltpu.VMEM((_GCH, _GW), jnp.int32),
            pltpu.VMEM((_GCH, _GW), jnp.int32),
            pltpu.SemaphoreType.DMA,
            pltpu.SemaphoreType.DMA,
            pltpu.SemaphoreType.DMA,
            pltpu.SemaphoreType.DMA,
            pltpu.SemaphoreType.DMA,
            pltpu.SemaphoreType.DMA,
            pltpu.SemaphoreType.DMA,
            pltpu.SemaphoreType.DMA,
        ],
    )
    def k(xs_hbm, xo_hbm, s_hbm, o_hbm, outs_hbm, outo_hbm, idx_s, idx_o,
          bs0, bo0, bs1, bo1, sgs0, sgo0, sgs1, sgo1, sws0, swo0, sws1, swo1):
        wid = lax.axis_index("s") * nc + lax.axis_index("c")
        base = wid * per_w
        pltpu.sync_copy(s_hbm.at[pl.ds(base, per_w)], idx_s)
        pltpu.sync_copy(o_hbm.at[pl.ds(base, per_w)], idx_o)

        bufs = ((bs0, bo0, sgs0, sgo0, sws0, swo0),
                (bs1, bo1, sgs1, sgo1, sws1, swo1))

        def start(c, b):
            bs, bo, sgs, sgo, _, _ = bufs[b]
            off = c * _GCH
            pltpu.async_copy(xs_hbm.at[idx_s.at[pl.ds(off, _GCH)]], bs, sgs)
            pltpu.async_copy(xo_hbm.at[idx_o.at[pl.ds(off, _GCH)]], bo, sgo)

        def compute(c, b):
            bs, bo, sgs, sgo, sws, swo = bufs[b]
            off = c * _GCH
            pltpu.make_async_copy(
                xs_hbm.at[idx_s.at[pl.ds(off, _GCH)]], bs, sgs).wait()
            pltpu.make_async_copy(
                xo_hbm.at[idx_o.at[pl.ds(off, _GCH)]], bo, sgo).wait()
            pltpu.async_copy(bs, outs_hbm.at[pl.ds(base + off, _GCH)], sws)
            pltpu.async_copy(bo, outo_hbm.at[pl.ds(base + off, _GCH)], swo)

        def drain_w(c, b):
            bs, bo, _, _, sws, swo = bufs[b]
            off = base + c * _GCH
            pltpu.make_async_copy(
                bs, outs_hbm.at[pl.ds(off, _GCH)], sws).wait()
            pltpu.make_async_copy(
                bo, outo_hbm.at[pl.ds(off, _GCH)], swo).wait()

        start(0, 0)
        start(1, 1)

        def pair(j, carry):
            c0 = 2 * j
            compute(c0, 0)
            compute(c0 + 1, 1)
            drain_w(c0, 0)
            start(c0 + 2, 0)
            drain_w(c0 + 1, 1)
            start(c0 + 3, 1)
            return carry

        lax.fori_loop(0, npairs, pair, 0)
        for c in range(2 * npairs, n_ch):
            b = c % 2
            compute(c, b)
            drain_w(c, b)
            if c + 2 < n_ch:
                start(c + 2, b)

    return k(xs, xo, s_idx, o_idx)


def _sc_scatter(us, uo, s_idx, o_idx, n_obj):
    n_edge = s_idx.shape[0]
    info = plsc.get_sparse_core_info()
    nc, ns = info.num_cores, info.num_subcores
    per_t = n_edge // ns
    n_ch = per_t // _SCH
    n_pad = _pad_nodes(n_obj, ns)
    stripe = n_pad // ns
    nz = stripe // _ZR
    n_sl = HID // 128
    sl_per_core = n_sl // nc
    assert per_t % _SCH == 0 and stripe % _ZR == 0
    mesh = plsc.VectorSubcoreMesh(core_axis_name="c", subcore_axis_name="s")

    assert n_ch >= 4
    npairs = (n_ch - 3) // 2

    @functools.partial(
        pl.kernel,
        out_type=jax.ShapeDtypeStruct((n_pad, HID), F32),
        mesh=mesh,
        scratch_types=[
            pltpu.VMEM((_SCH,), jnp.int32),
            pltpu.VMEM((_SCH,), jnp.int32),
            pltpu.VMEM((_SCH, 128), F32),
            pltpu.VMEM((_SCH, 128), F32),
            pltpu.VMEM((_ZR, 128), F32),
            pltpu.VMEM_SHARED((n_pad, 128), F32),
            pltpu.SemaphoreType.DMA,
            pltpu.SemaphoreType.DMA,
            pltpu.SemaphoreType.DMA,
            pltpu.SemaphoreType.DMA,
            pltpu.SemaphoreType.DMA,
            pltpu.SemaphoreType.DMA,
        ],
    )
    def k(us_hbm, uo_hbm, s_hbm, o_hbm, out_hbm, ib0, ib1, r0, r1,
          zero_b, acc, si0, si1, sr0, sr1, sa0, sa1):
        cid = lax.axis_index("c")
        sid = lax.axis_index("s")

        def zrow(r, c2):
            for cc in range(128 // 16):
                zero_b[r, pl.ds(cc * 16, 16)] = jnp.zeros((16,), F32)
            return c2

        lax.fori_loop(0, _ZR, zrow, 0)

        for f_local in range(sl_per_core):
            f = cid * sl_per_core + f_local
            for z in range(nz):
                pltpu.sync_copy(zero_b,
                                acc.at[pl.ds(sid * stripe + z * _ZR, _ZR)])
            plsc.subcore_barrier()

            for src_hbm, idx_hbm in ((us_hbm, s_hbm), (uo_hbm, o_hbm)):
                bufs = ((ib0, r0, si0, sr0, sa0), (ib1, r1, si1, sr1, sa1))

                def start(c, b):
                    ib, rb, si, sr, _ = bufs[b]
                    off = sid * per_t + c * _SCH
                    pltpu.async_copy(idx_hbm.at[pl.ds(off, _SCH)], ib, si)
                    pltpu.async_copy(
                        src_hbm.at[pl.ds(off, _SCH), pl.ds(f * 128, 128)],
                        rb, sr)

                def scat(c, b):
                    ib, rb, si, sr, sa = bufs[b]
                    off = sid * per_t + c * _SCH
                    pltpu.make_async_copy(
                        idx_hbm.at[pl.ds(off, _SCH)], ib, si).wait()
                    pltpu.make_async_copy(
                        src_hbm.at[pl.ds(off, _SCH), pl.ds(f * 128, 128)],
                        rb, sr).wait()
                    pltpu.async_copy(rb, acc.at[ib], sa, add=True)

                def drain_a(c, b):
                    ib, rb, _, _, sa = bufs[b]
                    pltpu.make_async_copy(rb, acc.at[ib], sa).wait()

                start(0, 0)
                start(1, 1)

                def pair(j, carry):
                    c0 = 2 * j
                    scat(c0, 0)
                    scat(c0 + 1, 1)
                    drain_a(c0, 0)
                    start(c0 + 2, 0)
                    drain_a(c0 + 1, 1)
                    start(c0 + 3, 1)
                    return carry

                lax.fori_loop(0, npairs, pair, 0)
                for c in range(2 * npairs, n_ch):
                    b = c % 2
                    scat(c, b)
                    drain_a(c, b)
                    if c + 2 < n_ch:
                        start(c + 2, b)

            plsc.subcore_barrier()
            pltpu.sync_copy(
                acc.at[pl.ds(sid * stripe, stripe)],
                out_hbm.at[pl.ds(sid * stripe, stripe), pl.ds(f * 128, 128)])
            plsc.subcore_barrier()

    return k(us, uo, s_idx, o_idx)


def _sc_counts(s_idx, o_idx, n_obj):
    n_edge = s_idx.shape[0]
    info = plsc.get_sparse_core_info()
    nc, ns = info.num_cores, info.num_subcores
    per_t = n_edge // ns
    n_ch = per_t // _SCH
    n_pad = _pad_nodes(n_obj, ns)
    stripe = n_pad // ns
    nz = stripe // _ZR
    assert per_t % _SCH == 0 and stripe % _ZR == 0 and nc == 2
    mesh = plsc.VectorSubcoreMesh(core_axis_name="c", subcore_axis_name="s")

    @functools.partial(
        pl.kernel,
        out_type=jax.ShapeDtypeStruct((nc * n_pad, 128), F32),
        mesh=mesh,
        scratch_types=[
            pltpu.VMEM((_SCH,), jnp.int32),
            pltpu.VMEM((_SCH, 128), F32),
            pltpu.VMEM((_ZR, 128), F32),
            pltpu.VMEM_SHARED((n_pad, 128), F32),
        ],
    )
    def k(s_hbm, o_hbm, out_hbm, idx_b, ones_b, zero_b, acc):
        cid = lax.axis_index("c")
        sid = lax.axis_index("s")

        def onerow(r, c2):
            for cc in range(128 // 16):
                ones_b[r, pl.ds(cc * 16, 16)] = jnp.ones((16,), F32)
            return c2

        lax.fori_loop(0, _SCH, onerow, 0)

        def zrow(r, c2):
            for cc in range(128 // 16):
                zero_b[r, pl.ds(cc * 16, 16)] = jnp.zeros((16,), F32)
            return c2

        lax.fori_loop(0, _ZR, zrow, 0)
        for z in range(nz):
            pltpu.sync_copy(zero_b, acc.at[pl.ds(sid * stripe + z * _ZR, _ZR)])
        plsc.subcore_barrier()

        def chunk(i, carry, idx_hbm):
            off = sid * per_t + i * _SCH
            pltpu.sync_copy(idx_hbm.at[pl.ds(off, _SCH)], idx_b)
            pltpu.sync_copy(ones_b, acc.at[idx_b], add=True)
            return carry

        @pl.when(cid == 0)
        def _():
            lax.fori_loop(0, n_ch, functools.partial(chunk, idx_hbm=s_hbm), 0)

        @pl.when(cid == 1)
        def _():
            lax.fori_loop(0, n_ch, functools.partial(chunk, idx_hbm=o_hbm), 0)

        plsc.subcore_barrier()
        pltpu.sync_copy(
            acc.at[pl.ds(sid * stripe, stripe)],
            out_hbm.at[pl.ds(cid * n_pad + sid * stripe, stripe)])

    return k(s_idx, o_idx)


def kernel(objs, triples, boxes_gt, attributes, enc_text_feat, enc_rel_feat,
           params):
    del attributes
    n_obj = objs.shape[0]
    s = triples[:, 0].astype(jnp.int32)
    p = triples[:, 1].astype(jnp.int32)
    o = triples[:, 2].astype(jnp.int32)

    w3, b3 = params["d3"]
    obj = _init_obj(objs, enc_text_feat, boxes_gt, params["obj_emb"], w3, b3)

    counts_flat = _sc_counts(s, o, n_obj)
    counts2 = counts_flat.reshape(2, -1, 128)

    e_feat = None
    for li, layer in enumerate(params["gconv"]):
        (w1, b1), (w2, b2) = layer["net1"]
        (v1, c1), (v2, c2) = layer["net2"]
        ws = w1[:DIN]
        wp = w1[DIN:2 * DIN]
        wo = w1[2 * DIN:]
        w_so = jnp.concatenate([ws, wo], axis=1)

        xsb, xo = _proj(obj, w_so, b1)
        gs, go = _sc_gather(xsb, xo, s, o)

        if li == 0:
            us, up, uo = _edge1(enc_rel_feat, p, gs, go, wp[:ADD],
                                params["pred_emb"], wp[ADD:], w2, b2)
        elif li == len(params["gconv"]) - 1:
            w2so = jnp.concatenate([w2[:, :HID], w2[:, HID + DIN:]], axis=1)
            b2so = jnp.concatenate([b2[:HID], b2[HID + DIN:]])
            us, uo = _edge_last(e_feat, gs, go, wp, w2so, b2so)
            up = None
        else:
            us, up, uo = _edge(e_feat, gs, go, wp, w2, b2)

        pooled = _sc_scatter(us, uo, s, o, n_obj)
        obj = _node(pooled, counts2, v1, c1, v2, c2, n_obj)
        e_feat = up

    mu, logvar = _heads(obj, params["mean_var"], params["mean"], params["var"])
    return mu, logvar


# speedup vs baseline: 3.2760x; 1.1288x over previous
import functools

import jax
import jax.numpy as jnp
from jax import lax
from jax.experimental import pallas as pl
from jax.experimental.pallas import tpu as pltpu
from jax.experimental.pallas import tpu_sc as plsc

EMB = 128
ADD = 512
HID = 512
DIN = 2 * EMB + ADD
N_PRED = 26

F32 = jnp.float32
BF16 = jnp.bfloat16
U32 = jnp.uint32
_GW = HID // 2


def _pack_bf16(v):
    lo = lax.bitcast_convert_type(v[:, :_GW], U32)
    hi = lax.bitcast_convert_type(v[:, _GW:], U32)

    def rne(u):
        return u + jnp.asarray(0x7FFF, U32) + ((u >> 16) & jnp.asarray(1, U32))

    packed = (rne(hi) & jnp.asarray(0xFFFF0000, U32)) | (rne(lo) >> 16)
    return lax.bitcast_convert_type(packed, jnp.int32)


def _unpack_bf16(gi):
    u = lax.bitcast_convert_type(gi, U32)
    lo = lax.bitcast_convert_type(u << 16, F32)
    hi = lax.bitcast_convert_type(u & jnp.asarray(0xFFFF0000, U32), F32)
    return jnp.concatenate([lo, hi], axis=1)


def _init_obj_body(objs_ref, et_ref, bx_ref, emb_ref, w3_ref, b3_ref, out_ref):
    out_ref[:, :ADD] = et_ref[...]
    objs = objs_ref[...]
    ncls = emb_ref.shape[0]
    oh = (objs == lax.broadcasted_iota(jnp.int32, (objs.shape[0], ncls), 1))
    out_ref[:, ADD:ADD + EMB] = jnp.dot(oh.astype(F32), emb_ref[...],
                                        preferred_element_type=F32)
    out_ref[:, ADD + EMB:] = (
        jnp.dot(bx_ref[...], w3_ref[...], preferred_element_type=F32)
        + b3_ref[...])


def _init_obj(objs, enc_text, boxes, emb, w3, b3):
    n = objs.shape[0]
    br = 1000
    ncls = emb.shape[0]
    return pl.pallas_call(
        _init_obj_body,
        grid=(n // br,),
        in_specs=[
            pl.BlockSpec((br, 1), lambda i: (i, 0)),
            pl.BlockSpec((br, ADD), lambda i: (i, 0)),
            pl.BlockSpec((br, 6), lambda i: (i, 0)),
            pl.BlockSpec((ncls, EMB), lambda i: (0, 0)),
            pl.BlockSpec((6, EMB), lambda i: (0, 0)),
            pl.BlockSpec((1, EMB), lambda i: (0, 0)),
        ],
        out_specs=pl.BlockSpec((br, DIN), lambda i: (i, 0)),
        out_shape=jax.ShapeDtypeStruct((n, DIN), F32),
    )(objs.reshape(n, 1).astype(jnp.int32), enc_text, boxes, emb, w3,
      b3.reshape(1, EMB))


def _proj_body(x_ref, w_ref, b_ref, xs_ref, xo_ref):
    y = jnp.dot(x_ref[...], w_ref[...], preferred_element_type=F32)
    xs_ref[...] = _pack_bf16(y[:, :HID] + b_ref[...])
    xo_ref[...] = _pack_bf16(y[:, HID:])


def _proj(x, w_so, b1):
    n = x.shape[0]
    br = 1000
    return pl.pallas_call(
        _proj_body,
        grid=(n // br,),
        in_specs=[
            pl.BlockSpec((br, DIN), lambda i: (i, 0)),
            pl.BlockSpec((DIN, 2 * HID), lambda i: (0, 0)),
            pl.BlockSpec((1, HID), lambda i: (0, 0)),
        ],
        out_specs=[
            pl.BlockSpec((br, _GW), lambda i: (i, 0)),
            pl.BlockSpec((br, _GW), lambda i: (i, 0)),
        ],
        out_shape=[jax.ShapeDtypeStruct((n, _GW), jnp.int32)] * 2,
    )(x, w_so, b1.reshape(1, HID))


def _edge_body(e_ref, gs_ref, go_ref, wp_ref, w2_ref, b2_ref, us_ref,
               up_ref, uo_ref):
    ep = jnp.dot(e_ref[...], wp_ref[...], preferred_element_type=F32)
    t = jnp.maximum(
        ep + _unpack_bf16(gs_ref[...]) + _unpack_bf16(go_ref[...]), 0.0)
    u = jnp.dot(t, w2_ref[...], preferred_element_type=F32) + b2_ref[...]
    u = jnp.maximum(u, 0.0)
    us_ref[...] = u[:, :HID]
    up_ref[...] = u[:, HID:HID + DIN]
    uo_ref[...] = u[:, HID + DIN:]


def _edge(e, gs, go, wp, w2, b2):
    n = e.shape[0]
    br = 640
    dout = 2 * HID + DIN
    return pl.pallas_call(
        _edge_body,
        grid=(n // br,),
        in_specs=[
            pl.BlockSpec((br, DIN), lambda i: (i, 0)),
            pl.BlockSpec((br, _GW), lambda i: (i, 0)),
            pl.BlockSpec((br, _GW), lambda i: (i, 0)),
            pl.BlockSpec((DIN, HID), lambda i: (0, 0)),
            pl.BlockSpec((HID, dout), lambda i: (0, 0)),
            pl.BlockSpec((1, dout), lambda i: (0, 0)),
        ],
        out_specs=[
            pl.BlockSpec((br, HID), lambda i: (i, 0)),
            pl.BlockSpec((br, DIN), lambda i: (i, 0)),
            pl.BlockSpec((br, HID), lambda i: (i, 0)),
        ],
        out_shape=[
            jax.ShapeDtypeStruct((n, HID), F32),
            jax.ShapeDtypeStruct((n, DIN), F32),
            jax.ShapeDtypeStruct((n, HID), F32),
        ],
    )(e, gs, go, wp, w2, b2.reshape(1, dout))


def _edge1_body(er_ref, p_ref, gs_ref, go_ref, wpa_ref, pemb_ref, wpb_ref,
                w2_ref, b2_ref, us_ref, up_ref, uo_ref):
    ep = jnp.dot(er_ref[...], wpa_ref[...], preferred_element_type=F32)
    tab = jnp.dot(pemb_ref[...], wpb_ref[...], preferred_element_type=F32)
    p = p_ref[...]
    oh = (p == lax.broadcasted_iota(jnp.int32, (p.shape[0], N_PRED), 1))
    ep = ep + jnp.dot(oh.astype(F32), tab, preferred_element_type=F32)
    t = jnp.maximum(
        ep + _unpack_bf16(gs_ref[...]) + _unpack_bf16(go_ref[...]), 0.0)
    u = jnp.dot(t, w2_ref[...], preferred_element_type=F32) + b2_ref[...]
    u = jnp.maximum(u, 0.0)
    us_ref[...] = u[:, :HID]
    up_ref[...] = u[:, HID:HID + DIN]
    uo_ref[...] = u[:, HID + DIN:]


def _edge1(enc_rel, p, gs, go, wpa, pemb, wpb, w2, b2):
    n = enc_rel.shape[0]
    br = 640
    dout = 2 * HID + DIN
    return pl.pallas_call(
        _edge1_body,
        grid=(n // br,),
        in_specs=[
            pl.BlockSpec((br, ADD), lambda i: (i, 0)),
            pl.BlockSpec((br, 1), lambda i: (i, 0)),
            pl.BlockSpec((br, _GW), lambda i: (i, 0)),
            pl.BlockSpec((br, _GW), lambda i: (i, 0)),
            pl.BlockSpec((ADD, HID), lambda i: (0, 0)),
            pl.BlockSpec((N_PRED, 2 * EMB), lambda i: (0, 0)),
            pl.BlockSpec((2 * EMB, HID), lambda i: (0, 0)),
            pl.BlockSpec((HID, dout), lambda i: (0, 0)),
            pl.BlockSpec((1, dout), lambda i: (0, 0)),
        ],
        out_specs=[
            pl.BlockSpec((br, HID), lambda i: (i, 0)),
            pl.BlockSpec((br, DIN), lambda i: (i, 0)),
            pl.BlockSpec((br, HID), lambda i: (i, 0)),
        ],
        out_shape=[
            jax.ShapeDtypeStruct((n, HID), F32),
            jax.ShapeDtypeStruct((n, DIN), F32),
            jax.ShapeDtypeStruct((n, HID), F32),
        ],
    )(enc_rel, p.reshape(n, 1).astype(jnp.int32), gs, go, wpa, pemb, wpb,
      w2, b2.reshape(1, dout))


def _edge_last_body(e_ref, gs_ref, go_ref, wp_ref, w2_ref, b2_ref, us_ref,
                    uo_ref):
    ep = jnp.dot(e_ref[...], wp_ref[...], preferred_element_type=F32)
    t = jnp.maximum(
        ep + _unpack_bf16(gs_ref[...]) + _unpack_bf16(go_ref[...]), 0.0)
    u = jnp.dot(t, w2_ref[...], preferred_element_type=F32) + b2_ref[...]
    u = jnp.maximum(u, 0.0)
    us_ref[...] = u[:, :HID]
    uo_ref[...] = u[:, HID:]


def _edge_last(e, gs, go, wp, w2so, b2so):
    n = e.shape[0]
    br = 640
    return pl.pallas_call(
        _edge_last_body,
        grid=(n // br,),
        in_specs=[
            pl.BlockSpec((br, DIN), lambda i: (i, 0)),
            pl.BlockSpec((br, _GW), lambda i: (i, 0)),
            pl.BlockSpec((br, _GW), lambda i: (i, 0)),
            pl.BlockSpec((DIN, HID), lambda i: (0, 0)),
            pl.BlockSpec((HID, 2 * HID), lambda i: (0, 0)),
            pl.BlockSpec((1, 2 * HID), lambda i: (0, 0)),
        ],
        out_specs=[
            pl.BlockSpec((br, HID), lambda i: (i, 0)),
            pl.BlockSpec((br, HID), lambda i: (i, 0)),
        ],
        out_shape=[jax.ShapeDtypeStruct((n, HID), F32)] * 2,
    )(e, gs, go, wp, w2so, b2so.reshape(1, 2 * HID))


def _node_body(pool_ref, cnt_ref, v1_ref, c1_ref, v2_ref, c2_ref, out_ref):
    cnt = cnt_ref[0, :, :1] + cnt_ref[1, :, :1]
    inv = 1.0 / jnp.maximum(cnt, 1.0)
    x = pool_ref[...] * inv
    h = jnp.maximum(
        jnp.dot(x, v1_ref[...], preferred_element_type=F32) + c1_ref[...], 0.0)
    out_ref[...] = jnp.maximum(
        jnp.dot(h, v2_ref[...], preferred_element_type=F32) + c2_ref[...], 0.0)


def _node(pooled, counts2, v1, c1, v2, c2, n):
    br = 1000
    return pl.pallas_call(
        _node_body,
        grid=(n // br,),
        in_specs=[
            pl.BlockSpec((br, HID), lambda i: (i, 0)),
            pl.BlockSpec((2, br, 128), lambda i: (0, i, 0)),
            pl.BlockSpec((HID, HID), lambda i: (0, 0)),
            pl.BlockSpec((1, HID), lambda i: (0, 0)),
            pl.BlockSpec((HID, DIN), lambda i: (0, 0)),
            pl.BlockSpec((1, DIN), lambda i: (0, 0)),
        ],
        out_specs=pl.BlockSpec((br, DIN), lambda i: (i, 0)),
        out_shape=jax.ShapeDtypeStruct((n, DIN), F32),
    )(pooled, counts2, v1, c1.reshape(1, HID), v2, c2.reshape(1, DIN))


def _heads_body(x_ref, m1_ref, d1_ref, m2_ref, d2_ref, wm_ref, bm_ref,
                wv_ref, bv_ref, mu_ref, lv_ref):
    h = jnp.maximum(
        jnp.dot(x_ref[...], m1_ref[...], preferred_element_type=F32)
        + d1_ref[...], 0.0)
    ov3 = jnp.maximum(
        jnp.dot(h, m2_ref[...], preferred_element_type=F32) + d2_ref[...], 0.0)
    mu_ref[...] = (jnp.dot(ov3, wm_ref[...], preferred_element_type=F32)
                   + bm_ref[...])
    lv_ref[...] = (jnp.dot(ov3, wv_ref[...], preferred_element_type=F32)
                   + bv_ref[...])


def _heads(x, mean_var, mean, var):
    n = x.shape[0]
    br = 1000
    (m1, d1), (m2, d2) = mean_var
    (wm, bm), = mean
    (wv, bv), = var
    return pl.pallas_call(
        _heads_body,
        grid=(n // br,),
        in_specs=[
            pl.BlockSpec((br, DIN), lambda i: (i, 0)),
            pl.BlockSpec((DIN, 4 * EMB), lambda i: (0, 0)),
            pl.BlockSpec((1, 4 * EMB), lambda i: (0, 0)),
            pl.BlockSpec((4 * EMB, 2 * EMB), lambda i: (0, 0)),
            pl.BlockSpec((1, 2 * EMB), lambda i: (0, 0)),
            pl.BlockSpec((2 * EMB, EMB), lambda i: (0, 0)),
            pl.BlockSpec((1, EMB), lambda i: (0, 0)),
            pl.BlockSpec((2 * EMB, EMB), lambda i: (0, 0)),
            pl.BlockSpec((1, EMB), lambda i: (0, 0)),
        ],
        out_specs=[
            pl.BlockSpec((br, EMB), lambda i: (i, 0)),
            pl.BlockSpec((br, EMB), lambda i: (i, 0)),
        ],
        out_shape=[jax.ShapeDtypeStruct((n, EMB), F32)] * 2,
    )(x, m1, d1.reshape(1, 4 * EMB), m2, d2.reshape(1, 2 * EMB),
      wm, bm.reshape(1, EMB), wv, bv.reshape(1, EMB))


_GCH = 40
_SCH = 80
_ZR = 128


def _pad_nodes(n_obj, ns):
    q = ns * _ZR
    return ((n_obj + q - 1) // q) * q


def _sc_gather(xs, xo, s_idx, o_idx):
    n_edge = s_idx.shape[0]
    info = plsc.get_sparse_core_info()
    nc, ns = info.num_cores, info.num_subcores
    nw = nc * ns
    per_w = n_edge // nw
    n_ch = per_w // _GCH
    assert per_w % _GCH == 0 and per_w % 8 == 0 and n_ch >= 6
    ntriples = (n_ch - 5) // 3
    mesh = plsc.VectorSubcoreMesh(core_axis_name="c", subcore_axis_name="s")

    @functools.partial(
        pl.kernel,
        out_type=[jax.ShapeDtypeStruct((n_edge, _GW), jnp.int32)] * 2,
        mesh=mesh,
        scratch_types=(
            [pltpu.VMEM((per_w,), jnp.int32)] * 2
            + [pltpu.VMEM((_GCH, _GW), jnp.int32)] * 6
            + [pltpu.SemaphoreType.DMA] * 12
        ),
    )
    def k(xs_hbm, xo_hbm, s_hbm, o_hbm, outs_hbm, outo_hbm, idx_s, idx_o,
          bs0, bs1, bs2, bo0, bo1, bo2,
          sgs0, sgs1, sgs2, sgo0, sgo1, sgo2,
          sws0, sws1, sws2, swo0, swo1, swo2):
        wid = lax.axis_index("s") * nc + lax.axis_index("c")
        base = wid * per_w
        pltpu.sync_copy(s_hbm.at[pl.ds(base, per_w)], idx_s)
        pltpu.sync_copy(o_hbm.at[pl.ds(base, per_w)], idx_o)

        bufs = ((bs0, bo0, sgs0, sgo0, sws0, swo0),
                (bs1, bo1, sgs1, sgo1, sws1, swo1),
                (bs2, bo2, sgs2, sgo2, sws2, swo2))

        def start(c, b):
            bs, bo, sgs, sgo, _, _ = bufs[b]
            off = c * _GCH
            pltpu.async_copy(xs_hbm.at[idx_s.at[pl.ds(off, _GCH)]], bs, sgs)
            pltpu.async_copy(xo_hbm.at[idx_o.at[pl.ds(off, _GCH)]], bo, sgo)

        def compute(c, b):
            bs, bo, sgs, sgo, sws, swo = bufs[b]
            off = c * _GCH
            pltpu.make_async_copy(
                xs_hbm.at[idx_s.at[pl.ds(off, _GCH)]], bs, sgs).wait()
            pltpu.make_async_copy(
                xo_hbm.at[idx_o.at[pl.ds(off, _GCH)]], bo, sgo).wait()
            pltpu.async_copy(bs, outs_hbm.at[pl.ds(base + off, _GCH)], sws)
            pltpu.async_copy(bo, outo_hbm.at[pl.ds(base + off, _GCH)], swo)

        def drain_w(c, b):
            bs, bo, _, _, sws, swo = bufs[b]
            off = base + c * _GCH
            pltpu.make_async_copy(
                bs, outs_hbm.at[pl.ds(off, _GCH)], sws).wait()
            pltpu.make_async_copy(
                bo, outo_hbm.at[pl.ds(off, _GCH)], swo).wait()

        start(0, 0)
        start(1, 1)
        start(2, 2)

        def triple(j, carry):
            c0 = 3 * j
            compute(c0, 0)
            compute(c0 + 1, 1)
            compute(c0 + 2, 2)
            drain_w(c0, 0)
            start(c0 + 3, 0)
            drain_w(c0 + 1, 1)
            start(c0 + 4, 1)
            drain_w(c0 + 2, 2)
            start(c0 + 5, 2)
            return carry

        lax.fori_loop(0, ntriples, triple, 0)
        for c in range(3 * ntriples, n_ch):
            b = c % 3
            compute(c, b)
            drain_w(c, b)
            if c + 3 < n_ch:
                start(c + 3, b)

    return k(xs, xo, s_idx, o_idx)


def _sc_scatter(us, uo, s_idx, o_idx, n_obj):
    n_edge = s_idx.shape[0]
    info = plsc.get_sparse_core_info()
    nc, ns = info.num_cores, info.num_subcores
    per_t = n_edge // ns
    n_ch = per_t // _SCH
    n_pad = _pad_nodes(n_obj, ns)
    stripe = n_pad // ns
    nz = stripe // _ZR
    n_sl = HID // 128
    sl_per_core = n_sl // nc
    assert per_t % _SCH == 0 and stripe % _ZR == 0
    mesh = plsc.VectorSubcoreMesh(core_axis_name="c", subcore_axis_name="s")

    assert n_ch >= 4
    npairs = (n_ch - 3) // 2

    @functools.partial(
        pl.kernel,
        out_type=jax.ShapeDtypeStruct((n_pad, HID), F32),
        mesh=mesh,
        scratch_types=(
            [pltpu.VMEM((_SCH,), jnp.int32)] * 4
            + [pltpu.VMEM((_SCH, 128), F32)] * 4
            + [pltpu.VMEM_SHARED((n_pad, 128), F32)]
            + [pltpu.SemaphoreType.DMA] * 12
        ),
    )
    def k(us_hbm, uo_hbm, s_hbm, o_hbm, out_hbm,
          ibs0, ibs1, ibo0, ibo1, rs0, rs1, ro0, ro1, acc,
          sis0, sis1, sio0, sio1, srs0, srs1, sro0, sro1,
          sas0, sas1, sao0, sao1):
        cid = lax.axis_index("c")
        sid = lax.axis_index("s")
        nzc = stripe // _SCH
        assert stripe % _SCH == 0

        for f_local in range(sl_per_core):
            f = cid * sl_per_core + f_local

            def zrow(r, c2):
                for cc in range(128 // 16):
                    rs0[r, pl.ds(cc * 16, 16)] = jnp.zeros((16,), F32)
                return c2

            lax.fori_loop(0, _SCH, zrow, 0)
            for z in range(nzc):
                pltpu.sync_copy(rs0,
                                acc.at[pl.ds(sid * stripe + z * _SCH, _SCH)])
            plsc.subcore_barrier()

            bufs = ((ibs0, rs0, sis0, srs0, sas0, us_hbm, s_hbm),
                    (ibs1, rs1, sis1, srs1, sas1, us_hbm, s_hbm),
                    (ibo0, ro0, sio0, sro0, sao0, uo_hbm, o_hbm),
                    (ibo1, ro1, sio1, sro1, sao1, uo_hbm, o_hbm))

            def start(c, b):
                ib, rb, si, sr, _, src_hbm, idx_hbm = bufs[b]
                off = sid * per_t + c * _SCH
                pltpu.async_copy(idx_hbm.at[pl.ds(off, _SCH)], ib, si)
                pltpu.async_copy(
                    src_hbm.at[pl.ds(off, _SCH), pl.ds(f * 128, 128)],
                    rb, sr)

            def scat(c, b):
                ib, rb, si, sr, sa, src_hbm, idx_hbm = bufs[b]
                off = sid * per_t + c * _SCH
                pltpu.make_async_copy(
                    idx_hbm.at[pl.ds(off, _SCH)], ib, si).wait()
                pltpu.make_async_copy(
                    src_hbm.at[pl.ds(off, _SCH), pl.ds(f * 128, 128)],
                    rb, sr).wait()
                pltpu.async_copy(rb, acc.at[ib], sa, add=True)

            def drain_a(c, b):
                ib, rb, _, _, sa, _, _ = bufs[b]
                pltpu.make_async_copy(rb, acc.at[ib], sa).wait()

            start(0, 0)
            start(0, 2)
            start(1, 1)
            start(1, 3)

            def pair(j, carry):
                c0 = 2 * j
                scat(c0, 0)
                scat(c0, 2)
                scat(c0 + 1, 1)
                scat(c0 + 1, 3)
                drain_a(c0, 0)
                start(c0 + 2, 0)
                drain_a(c0, 2)
                start(c0 + 2, 2)
                drain_a(c0 + 1, 1)
                start(c0 + 3, 1)
                drain_a(c0 + 1, 3)
                start(c0 + 3, 3)
                return carry

            lax.fori_loop(0, npairs, pair, 0)
            for c in range(2 * npairs, n_ch):
                b = c % 2
                scat(c, b)
                scat(c, b + 2)
                drain_a(c, b)
                drain_a(c, b + 2)
                if c + 2 < n_ch:
                    start(c + 2, b)
                    start(c + 2, b + 2)

            plsc.subcore_barrier()
            pltpu.sync_copy(
                acc.at[pl.ds(sid * stripe, stripe)],
                out_hbm.at[pl.ds(sid * stripe, stripe), pl.ds(f * 128, 128)])
            plsc.subcore_barrier()

    return k(us, uo, s_idx, o_idx)


def _sc_counts(s_idx, o_idx, n_obj):
    n_edge = s_idx.shape[0]
    info = plsc.get_sparse_core_info()
    nc, ns = info.num_cores, info.num_subcores
    per_t = n_edge // ns
    n_ch = per_t // _SCH
    n_pad = _pad_nodes(n_obj, ns)
    stripe = n_pad // ns
    nz = stripe // _ZR
    assert per_t % _SCH == 0 and stripe % _ZR == 0 and nc == 2
    mesh = plsc.VectorSubcoreMesh(core_axis_name="c", subcore_axis_name="s")

    @functools.partial(
        pl.kernel,
        out_type=jax.ShapeDtypeStruct((nc * n_pad, 128), F32),
        mesh=mesh,
        scratch_types=[
            pltpu.VMEM((_SCH,), jnp.int32),
            pltpu.VMEM((_SCH, 128), F32),
            pltpu.VMEM((_ZR, 128), F32),
            pltpu.VMEM_SHARED((n_pad, 128), F32),
        ],
    )
    def k(s_hbm, o_hbm, out_hbm, idx_b, ones_b, zero_b, acc):
        cid = lax.axis_index("c")
        sid = lax.axis_index("s")

        def onerow(r, c2):
            for cc in range(128 // 16):
                ones_b[r, pl.ds(cc * 16, 16)] = jnp.ones((16,), F32)
            return c2

        lax.fori_loop(0, _SCH, onerow, 0)

        def zrow(r, c2):
            for cc in range(128 // 16):
                zero_b[r, pl.ds(cc * 16, 16)] = jnp.zeros((16,), F32)
            return c2

        lax.fori_loop(0, _ZR, zrow, 0)
        for z in range(nz):
            pltpu.sync_copy(zero_b, acc.at[pl.ds(sid * stripe + z * _ZR, _ZR)])
        plsc.subcore_barrier()

        def chunk(i, carry, idx_hbm):
            off = sid * per_t + i * _SCH
            pltpu.sync_copy(idx_hbm.at[pl.ds(off, _SCH)], idx_b)
            pltpu.sync_copy(ones_b, acc.at[idx_b], add=True)
            return carry

        @pl.when(cid == 0)
        def _():
            lax.fori_loop(0, n_ch, functools.partial(chunk, idx_hbm=s_hbm), 0)

        @pl.when(cid == 1)
        def _():
            lax.fori_loop(0, n_ch, functools.partial(chunk, idx_hbm=o_hbm), 0)

        plsc.subcore_barrier()
        pltpu.sync_copy(
            acc.at[pl.ds(sid * stripe, stripe)],
            out_hbm.at[pl.ds(cid * n_pad + sid * stripe, stripe)])

    return k(s_idx, o_idx)


def kernel(objs, triples, boxes_gt, attributes, enc_text_feat, enc_rel_feat,
           params):
    del attributes
    n_obj = objs.shape[0]
    s = triples[:, 0].astype(jnp.int32)
    p = triples[:, 1].astype(jnp.int32)
    o = triples[:, 2].astype(jnp.int32)

    w3, b3 = params["d3"]
    obj = _init_obj(objs, enc_text_feat, boxes_gt, params["obj_emb"], w3, b3)

    counts_flat = _sc_counts(s, o, n_obj)
    counts2 = counts_flat.reshape(2, -1, 128)

    e_feat = None
    for li, layer in enumerate(params["gconv"]):
        (w1, b1), (w2, b2) = layer["net1"]
        (v1, c1), (v2, c2) = layer["net2"]
        ws = w1[:DIN]
        wp = w1[DIN:2 * DIN]
        wo = w1[2 * DIN:]
        w_so = jnp.concatenate([ws, wo], axis=1)

        xsb, xo = _proj(obj, w_so, b1)
        gs, go = _sc_gather(xsb, xo, s, o)

        if li == 0:
            us, up, uo = _edge1(enc_rel_feat, p, gs, go, wp[:ADD],
                                params["pred_emb"], wp[ADD:], w2, b2)
        elif li == len(params["gconv"]) - 1:
            w2so = jnp.concatenate([w2[:, :HID], w2[:, HID + DIN:]], axis=1)
            b2so = jnp.concatenate([b2[:HID], b2[HID + DIN:]])
            us, uo = _edge_last(e_feat, gs, go, wp, w2so, b2so)
            up = None
        else:
            us, up, uo = _edge(e_feat, gs, go, wp, w2, b2)

        pooled = _sc_scatter(us, uo, s, o, n_obj)
        obj = _node(pooled, counts2, v1, c1, v2, c2, n_obj)
        e_feat = up

    mu, logvar = _heads(obj, params["mean_var"], params["mean"], params["var"])
    return mu, logvar
